# baseline (speedup 1.0000x reference)
import functools

import jax
import jax.numpy as jnp
from jax import lax
from jax.experimental import pallas as pl
from jax.experimental.pallas import tpu as pltpu
from jax.experimental.pallas import tpu_sc as plsc

D = 64
SCALE = 8.0
LANES = 16
CH = 128
C = 256
NBUF = 4
K = C // CH


def _build_sc_kernel(B: int, V: int):
    info = plsc.get_sparse_core_info()
    NC, NS = info.num_cores, info.num_subcores
    NW = NC * NS
    per_w = B // NW
    n_idx_rows = per_w // CH
    G = per_w // C
    S = G // NBUF
    assert B % (NW * C) == 0 and G % NBUF == 0 and S >= 3

    mesh = plsc.VectorSubcoreMesh(core_axis_name="c", subcore_axis_name="s")

    @functools.partial(
        pl.kernel,
        mesh=mesh,
        out_type=jax.ShapeDtypeStruct((B, D), jnp.float32),
        scratch_types=[
            pltpu.VMEM((n_idx_rows, CH), jnp.int32),
            *[pltpu.VMEM((C, D), jnp.float32) for _ in range(NBUF)],
            *[pltpu.SemaphoreType.DMA for _ in range(2 * NBUF)],
        ],
        compiler_params=pltpu.CompilerParams(use_tc_tiling_on_sc=False),
    )
    def k(x_hbm, table_hbm, out_hbm, idx_v, *bufs_and_sems):
        rows = bufs_and_sems[:NBUF]
        gsem = bufs_and_sems[NBUF:2 * NBUF]
        osem = bufs_and_sems[2 * NBUF:]

        wid = lax.axis_index("s") * NC + lax.axis_index("c")
        base = wid * per_w

        pltpu.sync_copy(x_hbm.at[pl.ds(wid * n_idx_rows, n_idx_rows)], idx_v)

        def fire_gather(g, b):
            for j in range(K):
                pltpu.async_copy(
                    table_hbm.at[idx_v.at[g * K + j]],
                    rows[b].at[pl.ds(j * CH, CH)],
                    gsem[b],
                )

        def drain_gather(b):
            pltpu.make_async_copy(
                table_hbm.at[pl.ds(0, C)], rows[b], gsem[b]).wait()

        def wait_out(b):
            pltpu.make_async_copy(
                rows[b], out_hbm.at[pl.ds(0, C)], osem[b]).wait()

        def scale_buf(b):
            def body(i, carry):
                for j in range(D // LANES):
                    sl = (i, pl.ds(j * LANES, LANES))
                    rows[b][sl] = rows[b][sl] * SCALE
                return carry
            lax.fori_loop(0, C, body, 0)

        def step(g, b, first, last):
            drain_gather(b)
            scale_buf(b)
            pltpu.async_copy(
                rows[b], out_hbm.at[pl.ds(base + g * C, C)], osem[b])
            nb = (b + 2) % NBUF
            if not last:
                if not first:
                    wait_out(nb)
                fire_gather(g + 2, nb)
            elif not first:
                wait_out(nb)

        fire_gather(0, 0)
        fire_gather(1, 1)

        for b in range(NBUF):
            step(b, b, first=(b < 2), last=False)

        def rev(s, carry):
            for b in range(NBUF):
                step(s * NBUF + b, b, first=False, last=False)
            return carry
        lax.fori_loop(1, S - 1, rev, 0)

        for b in range(NBUF):
            g = (S - 1) * NBUF + b
            step(g, b, first=False, last=(b >= 2))
        wait_out(2)
        wait_out(3)

    return k


def kernel(x, table):
    B = x.shape[0] * x.shape[1]
    V = table.shape[0]
    x_flat = x.reshape(B // CH, CH).astype(jnp.int32)
    out = _build_sc_kernel(B, V)(x_flat, table)
    return out.reshape(x.shape[0], x.shape[1], D)
